# baseline (speedup 1.0000x reference)
import functools

import jax
import jax.numpy as jnp
from jax import lax
from jax.experimental import pallas as pl
from jax.experimental.pallas import tpu as pltpu
from jax.experimental.pallas import tpu_sc as plsc


def _ffn_body(xe_ref, w1_ref, b1_ref, w2_ref, b2_ref, g_ref, *rest):
    out_ref = rest[-2]
    acc_ref = rest[-1]
    j = pl.program_id(1)
    nj = pl.num_programs(1)
    xb = xe_ref[0].astype(jnp.bfloat16)
    w1b = w1_ref[0].astype(jnp.bfloat16)
    h = jax.lax.dot_general(
        xb, w1b, (((1,), (1,)), ((), ())),
        preferred_element_type=jnp.float32)
    h = h + b1_ref[0, 0, 0][None, :]
    h = 0.5 * h * (1.0 + jax.lax.erf(h * 0.7071067811865476))
    w2b = w2_ref[0].astype(jnp.bfloat16)
    part = jax.lax.dot_general(
        h.astype(jnp.bfloat16), w2b, (((1,), (1,)), ((), ())),
        preferred_element_type=jnp.float32)

    @pl.when(j == 0)
    def _():
        acc_ref[...] = part

    @pl.when(j != 0)
    def _():
        acc_ref[...] += part

    @pl.when(j == nj - 1)
    def _():
        out_ref[0] = (acc_ref[...] + b2_ref[0, 0][None, :]) * g_ref[0, 0][:, None]


def _ffn_half(xe_h, w1, b1r, w2, b2r, gr_h, carry, *, dt, e0):
    eh, c, d = xe_h.shape
    e = w1.shape[0]
    dff = w1.shape[1]
    nj = dff // dt
    in_specs = [
        pl.BlockSpec((1, c, d), lambda i, j: (i, 0, 0)),
        pl.BlockSpec((1, dt, d), lambda i, j, e0=e0: (i + e0, j, 0)),
        pl.BlockSpec((1, 1, 1, dt), lambda i, j, e0=e0: (i + e0, j, 0, 0)),
        pl.BlockSpec((1, d, dt), lambda i, j, e0=e0: (i + e0, 0, j)),
        pl.BlockSpec((1, 1, d), lambda i, j, e0=e0: (i + e0, 0, 0)),
        pl.BlockSpec((1, 1, c), lambda i, j: (i, 0, 0)),
    ]
    args = [xe_h, w1, b1r, w2, b2r, gr_h]
    aliases = {}
    if carry is not None:
        in_specs.append(pl.BlockSpec(memory_space=pl.ANY))
        args.append(carry)
        aliases = {6: 0}
    return pl.pallas_call(
        _ffn_body,
        grid=(eh, nj),
        in_specs=in_specs,
        out_specs=pl.BlockSpec((1, c, d), lambda i, j, e0=e0: (i + e0, 0, 0)),
        out_shape=jax.ShapeDtypeStruct((e, c, d), jnp.float32),
        scratch_shapes=[pltpu.VMEM((c, d), jnp.float32)],
        input_output_aliases=aliases,
        compiler_params=pltpu.CompilerParams(
            dimension_semantics=("arbitrary", "arbitrary"),
        ),
    )(*args)


def _sc_gather_rows(table, idx, *, chunk=32):
    v, w = table.shape
    bsz = idx.shape[0]
    info = plsc.get_sparse_core_info()
    nw = info.num_cores * info.num_subcores
    b_per_w = bsz // nw
    assert b_per_w * nw == bsz and b_per_w % chunk == 0
    nch = b_per_w // chunk
    mesh = plsc.VectorSubcoreMesh(core_axis_name="c", subcore_axis_name="s")

    out_type = [jax.ShapeDtypeStruct((bsz, w), jnp.float32)]
    scratch = [
        pltpu.VMEM((b_per_w,), jnp.int32),
        pltpu.VMEM((chunk, w), jnp.float32),
        pltpu.VMEM((chunk, w), jnp.float32),
        pltpu.SemaphoreType.DMA,
        pltpu.SemaphoreType.DMA,
        pltpu.SemaphoreType.DMA,
        pltpu.SemaphoreType.DMA,
    ]
    @functools.partial(pl.kernel, mesh=mesh, out_type=tuple(out_type),
                       scratch_types=scratch)
    def k(*refs):
        (table_hbm, idx_hbm, out_hbm, idx_v,
         r0, r1, sg0, sg1, sw0, sw1) = refs
        bufs = (r0, r1)
        gsems = (sg0, sg1)
        wsems = (sw0, sw1)
        wid = lax.axis_index("s") * info.num_cores + lax.axis_index("c")
        base = wid * b_per_w
        pltpu.sync_copy(idx_hbm.at[pl.ds(base, b_per_w)], idx_v)

        def start_gather(c):
            bi = c % 2
            return [pltpu.async_copy(
                table_hbm.at[idx_v.at[pl.ds(c * chunk, chunk)]],
                bufs[bi], gsems[bi])]

        def start_wb(c):
            bi = c % 2
            off = base + c * chunk
            return [pltpu.async_copy(
                bufs[bi], out_hbm.at[pl.ds(off, chunk)], wsems[bi])]

        pending_g = start_gather(0)
        pending_w = [None, None]
        for c in range(nch):
            bi = c % 2
            for cp in pending_g:
                cp.wait()
            if c + 1 < nch:
                if pending_w[1 - bi] is not None:
                    for cp in pending_w[1 - bi]:
                        cp.wait()
                    pending_w[1 - bi] = None
                pending_g = start_gather(c + 1)
            pending_w[bi] = start_wb(c)
        for pw in pending_w:
            if pw is not None:
                for cp in pw:
                    cp.wait()

    return k(table, idx)[0]


def _sc_return_scatter(eo_flat, src3, dst3, n_out, *, chunk=32):
    v, w = eo_flat.shape
    nw, nch, ck = src3.shape
    assert ck == chunk
    mesh = plsc.VectorSubcoreMesh(core_axis_name="c", subcore_axis_name="s")

    @functools.partial(
        pl.kernel, mesh=mesh,
        out_type=jax.ShapeDtypeStruct((n_out, w), jnp.float32),
        scratch_types=[
            pltpu.VMEM((nch, chunk), jnp.int32),
            pltpu.VMEM((nch, chunk), jnp.int32),
            pltpu.VMEM((chunk, w), jnp.float32),
            pltpu.VMEM((chunk, w), jnp.float32),
            pltpu.SemaphoreType.DMA,
            pltpu.SemaphoreType.DMA,
            pltpu.SemaphoreType.DMA,
            pltpu.SemaphoreType.DMA,
        ],
    )
    def k(eo_hbm, src_hbm, dst_hbm, out_hbm, src_v, dst_v,
          r0, r1, sg0, sg1, sw0, sw1):
        bufs = (r0, r1)
        gsems = (sg0, sg1)
        wsems = (sw0, sw1)
        wid = lax.axis_index("s") * 2 + lax.axis_index("c")
        pltpu.sync_copy(src_hbm.at[wid], src_v)
        pltpu.sync_copy(dst_hbm.at[wid], dst_v)

        def start_gather(c):
            bi = c % 2
            return pltpu.async_copy(
                eo_hbm.at[src_v.at[c]], bufs[bi], gsems[bi])

        def start_wb(c):
            bi = c % 2
            return pltpu.async_copy(
                bufs[bi], out_hbm.at[dst_v.at[c]], wsems[bi])

        pending_g = start_gather(0)
        pending_w = [None, None]
        for c in range(nch):
            bi = c % 2
            pending_g.wait()
            if c + 1 < nch:
                if pending_w[1 - bi] is not None:
                    pending_w[1 - bi].wait()
                    pending_w[1 - bi] = None
                pending_g = start_gather(c + 1)
            pending_w[bi] = start_wb(c)
        for pw in pending_w:
            if pw is not None:
                pw.wait()

    return k(eo_flat, src3, dst3)


def kernel(x, Wr, w1, b1, w2, b2):
    b, s, d = x.shape
    e = Wr.shape[0]
    x_flat = x.reshape(-1, d)
    n_tok = x_flat.shape[0]
    cap = int(1.25 * n_tok / e)

    router_logits = x_flat @ Wr.T
    router_probs = jax.nn.softmax(router_logits, axis=-1)
    gates = jnp.max(router_probs, axis=-1)
    indices = jnp.argmax(router_probs, axis=-1)

    expert_mask = jax.nn.one_hot(indices, e, dtype=jnp.float32)
    density = expert_mask.mean(axis=0)
    prob_mean = router_probs.mean(axis=0)
    load_balance_loss = e * jnp.sum(density * prob_mean) * 0.01
    router_z_loss = jnp.mean(
        jax.scipy.special.logsumexp(router_probs, axis=-1)) * 0.001
    aux_loss = load_balance_loss + router_z_loss

    tok = jnp.arange(n_tok, dtype=jnp.int32)
    idx32 = indices.astype(jnp.int32)
    sorted_e, sorted_negg, sorted_tok = jax.lax.sort(
        (idx32, -gates, tok), num_keys=2, is_stable=True)
    counts = jnp.sum(expert_mask, axis=0).astype(jnp.int32)
    seg_start = jnp.concatenate(
        [jnp.zeros((1,), jnp.int32), jnp.cumsum(counts)[:-1].astype(jnp.int32)])
    pos = tok - seg_start[sorted_e]
    keep = pos < cap
    fslot = sorted_e * cap + pos
    f_or_dummy = jnp.where(keep, fslot, e * cap)
    sel2 = jnp.zeros((e * cap + 1, 2), jnp.float32).at[f_or_dummy].set(
        jnp.stack([sorted_tok.astype(jnp.float32), -sorted_negg], axis=1)
    )[: e * cap]
    sel_idx = sel2[:, 0].astype(jnp.int32)
    sel_gate = sel2[:, 1]

    half = e * cap // 2
    eh = e // 2
    xe0 = _sc_gather_rows(x_flat, sel_idx[:half], chunk=40).reshape(
        eh, cap, d)
    xe1 = _sc_gather_rows(x_flat, sel_idx[half:], chunk=40).reshape(
        eh, cap, d)
    gr = sel_gate.reshape(e, 1, cap)
    b1r = b1.reshape(e, -1, 1, 2048)
    b2r = b2.reshape(e, 1, d)

    eo_0 = _ffn_half(xe0, w1, b1r, w2, b2r, gr[:eh], None, dt=2048, e0=0)
    eo = _ffn_half(xe1, w1, b1r, w2, b2r, gr[eh:], eo_0, dt=2048, e0=eh)
    eo_flat = eo.reshape(e * cap, d)

    e_star = jnp.argmin(counts).astype(jnp.int32)
    pad_slot = e_star * cap + counts[e_star]
    src_sorted = jnp.where(keep, fslot, pad_slot)
    nw = 32
    src3 = src_sorted.reshape(nw, -1, 32)
    dst3 = sorted_tok.reshape(nw, -1, 32)
    out_flat = _sc_return_scatter(eo_flat, src3, dst3, n_tok)
    return out_flat.reshape(b, s, d), aux_loss


# speedup vs baseline: 1.0380x; 1.0380x over previous
import functools

import jax
import jax.numpy as jnp
from jax import lax
from jax.experimental import pallas as pl
from jax.experimental.pallas import tpu as pltpu
from jax.experimental.pallas import tpu_sc as plsc


def _ffn_body(xe_ref, w1_ref, b1_ref, w2_ref, b2_ref, g_ref, *rest):
    out_ref = rest[-2]
    acc_ref = rest[-1]
    j = pl.program_id(1)
    nj = pl.num_programs(1)
    xb = xe_ref[0].astype(jnp.bfloat16)
    w1b = w1_ref[0].astype(jnp.bfloat16)
    h = jax.lax.dot_general(
        xb, w1b, (((1,), (1,)), ((), ())),
        preferred_element_type=jnp.float32)
    h = h + b1_ref[0, 0, 0][None, :]
    h = 0.5 * h * (1.0 + jax.lax.erf(h * 0.7071067811865476))
    w2b = w2_ref[0].astype(jnp.bfloat16)
    part = jax.lax.dot_general(
        h.astype(jnp.bfloat16), w2b, (((1,), (1,)), ((), ())),
        preferred_element_type=jnp.float32)

    @pl.when(j == 0)
    def _():
        acc_ref[...] = part

    @pl.when(j != 0)
    def _():
        acc_ref[...] += part

    @pl.when(j == nj - 1)
    def _():
        out_ref[0] = (acc_ref[...] + b2_ref[0, 0][None, :]) * g_ref[0, 0][:, None]


def _ffn_half(xe_h, w1, b1r, w2, b2r, gr_h, carry, *, dt, e0):
    eh, c, d = xe_h.shape
    e = w1.shape[0]
    dff = w1.shape[1]
    nj = dff // dt
    in_specs = [
        pl.BlockSpec((1, c, d), lambda i, j: (i, 0, 0)),
        pl.BlockSpec((1, dt, d), lambda i, j, e0=e0: (i + e0, j, 0)),
        pl.BlockSpec((1, 1, 1, dt), lambda i, j, e0=e0: (i + e0, j, 0, 0)),
        pl.BlockSpec((1, d, dt), lambda i, j, e0=e0: (i + e0, 0, j)),
        pl.BlockSpec((1, 1, d), lambda i, j, e0=e0: (i + e0, 0, 0)),
        pl.BlockSpec((1, 1, c), lambda i, j: (i, 0, 0)),
    ]
    args = [xe_h, w1, b1r, w2, b2r, gr_h]
    aliases = {}
    if carry is not None:
        in_specs.append(pl.BlockSpec(memory_space=pl.ANY))
        args.append(carry)
        aliases = {6: 0}
    return pl.pallas_call(
        _ffn_body,
        grid=(eh, nj),
        in_specs=in_specs,
        out_specs=pl.BlockSpec((1, c, d), lambda i, j, e0=e0: (i + e0, 0, 0)),
        out_shape=jax.ShapeDtypeStruct((e, c, d), jnp.float32),
        scratch_shapes=[pltpu.VMEM((c, d), jnp.float32)],
        input_output_aliases=aliases,
        compiler_params=pltpu.CompilerParams(
            dimension_semantics=("parallel", "arbitrary"),
        ),
    )(*args)


def _sc_gather_rows(table, idx, *, chunk=32):
    v, w = table.shape
    bsz = idx.shape[0]
    info = plsc.get_sparse_core_info()
    nw = info.num_cores * info.num_subcores
    b_per_w = bsz // nw
    assert b_per_w * nw == bsz and b_per_w % chunk == 0
    nch = b_per_w // chunk
    mesh = plsc.VectorSubcoreMesh(core_axis_name="c", subcore_axis_name="s")

    out_type = [jax.ShapeDtypeStruct((bsz, w), jnp.float32)]
    scratch = [
        pltpu.VMEM((b_per_w,), jnp.int32),
        pltpu.VMEM((chunk, w), jnp.float32),
        pltpu.VMEM((chunk, w), jnp.float32),
        pltpu.SemaphoreType.DMA,
        pltpu.SemaphoreType.DMA,
        pltpu.SemaphoreType.DMA,
        pltpu.SemaphoreType.DMA,
    ]
    @functools.partial(pl.kernel, mesh=mesh, out_type=tuple(out_type),
                       scratch_types=scratch)
    def k(*refs):
        (table_hbm, idx_hbm, out_hbm, idx_v,
         r0, r1, sg0, sg1, sw0, sw1) = refs
        bufs = (r0, r1)
        gsems = (sg0, sg1)
        wsems = (sw0, sw1)
        wid = lax.axis_index("s") * info.num_cores + lax.axis_index("c")
        base = wid * b_per_w
        pltpu.sync_copy(idx_hbm.at[pl.ds(base, b_per_w)], idx_v)

        def start_gather(c):
            bi = c % 2
            return [pltpu.async_copy(
                table_hbm.at[idx_v.at[pl.ds(c * chunk, chunk)]],
                bufs[bi], gsems[bi])]

        def start_wb(c):
            bi = c % 2
            off = base + c * chunk
            return [pltpu.async_copy(
                bufs[bi], out_hbm.at[pl.ds(off, chunk)], wsems[bi])]

        pending_g = start_gather(0)
        pending_w = [None, None]
        for c in range(nch):
            bi = c % 2
            for cp in pending_g:
                cp.wait()
            if c + 1 < nch:
                if pending_w[1 - bi] is not None:
                    for cp in pending_w[1 - bi]:
                        cp.wait()
                    pending_w[1 - bi] = None
                pending_g = start_gather(c + 1)
            pending_w[bi] = start_wb(c)
        for pw in pending_w:
            if pw is not None:
                for cp in pw:
                    cp.wait()

    return k(table, idx)[0]


def _sc_return_scatter(eo_flat, src3, dst3, n_out, *, chunk=32):
    v, w = eo_flat.shape
    nw, nch, ck = src3.shape
    assert ck == chunk
    mesh = plsc.VectorSubcoreMesh(core_axis_name="c", subcore_axis_name="s")

    @functools.partial(
        pl.kernel, mesh=mesh,
        out_type=jax.ShapeDtypeStruct((n_out, w), jnp.float32),
        scratch_types=[
            pltpu.VMEM((nch, chunk), jnp.int32),
            pltpu.VMEM((nch, chunk), jnp.int32),
            pltpu.VMEM((chunk, w), jnp.float32),
            pltpu.VMEM((chunk, w), jnp.float32),
            pltpu.SemaphoreType.DMA,
            pltpu.SemaphoreType.DMA,
            pltpu.SemaphoreType.DMA,
            pltpu.SemaphoreType.DMA,
        ],
    )
    def k(eo_hbm, src_hbm, dst_hbm, out_hbm, src_v, dst_v,
          r0, r1, sg0, sg1, sw0, sw1):
        bufs = (r0, r1)
        gsems = (sg0, sg1)
        wsems = (sw0, sw1)
        wid = lax.axis_index("s") * 2 + lax.axis_index("c")
        pltpu.sync_copy(src_hbm.at[wid], src_v)
        pltpu.sync_copy(dst_hbm.at[wid], dst_v)

        def start_gather(c):
            bi = c % 2
            return pltpu.async_copy(
                eo_hbm.at[src_v.at[c]], bufs[bi], gsems[bi])

        def start_wb(c):
            bi = c % 2
            return pltpu.async_copy(
                bufs[bi], out_hbm.at[dst_v.at[c]], wsems[bi])

        pending_g = start_gather(0)
        pending_w = [None, None]
        for c in range(nch):
            bi = c % 2
            pending_g.wait()
            if c + 1 < nch:
                if pending_w[1 - bi] is not None:
                    pending_w[1 - bi].wait()
                    pending_w[1 - bi] = None
                pending_g = start_gather(c + 1)
            pending_w[bi] = start_wb(c)
        for pw in pending_w:
            if pw is not None:
                pw.wait()

    return k(eo_flat, src3, dst3)


def kernel(x, Wr, w1, b1, w2, b2):
    b, s, d = x.shape
    e = Wr.shape[0]
    x_flat = x.reshape(-1, d)
    n_tok = x_flat.shape[0]
    cap = int(1.25 * n_tok / e)

    router_logits = x_flat @ Wr.T
    router_probs = jax.nn.softmax(router_logits, axis=-1)
    gates = jnp.max(router_probs, axis=-1)
    indices = jnp.argmax(router_probs, axis=-1)

    expert_mask = jax.nn.one_hot(indices, e, dtype=jnp.float32)
    density = expert_mask.mean(axis=0)
    prob_mean = router_probs.mean(axis=0)
    load_balance_loss = e * jnp.sum(density * prob_mean) * 0.01
    router_z_loss = jnp.mean(
        jax.scipy.special.logsumexp(router_probs, axis=-1)) * 0.001
    aux_loss = load_balance_loss + router_z_loss

    tok = jnp.arange(n_tok, dtype=jnp.int32)
    idx32 = indices.astype(jnp.int32)
    sorted_e, sorted_negg, sorted_tok = jax.lax.sort(
        (idx32, -gates, tok), num_keys=2, is_stable=True)
    counts = jnp.sum(expert_mask, axis=0).astype(jnp.int32)
    seg_start = jnp.concatenate(
        [jnp.zeros((1,), jnp.int32), jnp.cumsum(counts)[:-1].astype(jnp.int32)])
    pos = tok - seg_start[sorted_e]
    keep = pos < cap
    fslot = sorted_e * cap + pos
    f_or_dummy = jnp.where(keep, fslot, e * cap)
    sel2 = jnp.zeros((e * cap + 1, 2), jnp.float32).at[f_or_dummy].set(
        jnp.stack([sorted_tok.astype(jnp.float32), -sorted_negg], axis=1)
    )[: e * cap]
    sel_idx = sel2[:, 0].astype(jnp.int32)
    sel_gate = sel2[:, 1]

    xe = _sc_gather_rows(x_flat, sel_idx, chunk=40).reshape(e, cap, d)
    gr = sel_gate.reshape(e, 1, cap)
    b1r = b1.reshape(e, -1, 1, 2048)
    b2r = b2.reshape(e, 1, d)

    eo = _ffn_half(xe, w1, b1r, w2, b2r, gr, None, dt=2048, e0=0)
    eo_flat = eo.reshape(e * cap, d)

    e_star = jnp.argmin(counts).astype(jnp.int32)
    pad_slot = e_star * cap + counts[e_star]
    src_sorted = jnp.where(keep, fslot, pad_slot)
    nw = 32
    src3 = src_sorted.reshape(nw, -1, 32)
    dst3 = sorted_tok.reshape(nw, -1, 32)
    out_flat = _sc_return_scatter(eo_flat, src3, dst3, n_tok)
    return out_flat.reshape(b, s, d), aux_loss


# speedup vs baseline: 1.4257x; 1.3734x over previous
import functools

import jax
import jax.numpy as jnp
from jax import lax
from jax.experimental import pallas as pl
from jax.experimental.pallas import tpu as pltpu
from jax.experimental.pallas import tpu_sc as plsc


def _ffn_body(xe_ref, w1_ref, b1_ref, w2_ref, b2_ref, g_ref, *rest):
    out_ref = rest[-2]
    acc_ref = rest[-1]
    j = pl.program_id(1)
    nj = pl.num_programs(1)
    xb = xe_ref[0].astype(jnp.bfloat16)
    w1b = w1_ref[0].astype(jnp.bfloat16)
    h = jax.lax.dot_general(
        xb, w1b, (((1,), (1,)), ((), ())),
        preferred_element_type=jnp.float32)
    h = h + b1_ref[0, 0, 0][None, :]
    h = 0.5 * h * (1.0 + jax.lax.erf(h * 0.7071067811865476))
    w2b = w2_ref[0].astype(jnp.bfloat16)
    part = jax.lax.dot_general(
        h.astype(jnp.bfloat16), w2b, (((1,), (1,)), ((), ())),
        preferred_element_type=jnp.float32)

    @pl.when(j == 0)
    def _():
        acc_ref[...] = part

    @pl.when(j != 0)
    def _():
        acc_ref[...] += part

    @pl.when(j == nj - 1)
    def _():
        g = g_ref[0, 0][:, None]
        out_ref[0] = jnp.where(
            g > 0.0, (acc_ref[...] + b2_ref[0, 0][None, :]) * g, 0.0)


def _ffn_half(xe_h, w1, b1r, w2, b2r, gr_h, carry, *, dt, e0):
    eh, c, d = xe_h.shape
    e = w1.shape[0]
    dff = w1.shape[1]
    nj = dff // dt
    in_specs = [
        pl.BlockSpec((1, c, d), lambda i, j: (i, 0, 0)),
        pl.BlockSpec((1, dt, d), lambda i, j, e0=e0: (i + e0, j, 0)),
        pl.BlockSpec((1, 1, 1, dt), lambda i, j, e0=e0: (i + e0, j, 0, 0)),
        pl.BlockSpec((1, d, dt), lambda i, j, e0=e0: (i + e0, 0, j)),
        pl.BlockSpec((1, 1, d), lambda i, j, e0=e0: (i + e0, 0, 0)),
        pl.BlockSpec((1, 1, c), lambda i, j: (i, 0, 0)),
    ]
    args = [xe_h, w1, b1r, w2, b2r, gr_h]
    aliases = {}
    if carry is not None:
        in_specs.append(pl.BlockSpec(memory_space=pl.ANY))
        args.append(carry)
        aliases = {6: 0}
    return pl.pallas_call(
        _ffn_body,
        grid=(eh, nj),
        in_specs=in_specs,
        out_specs=pl.BlockSpec((1, c, d), lambda i, j, e0=e0: (i + e0, 0, 0)),
        out_shape=jax.ShapeDtypeStruct((e, c, d), jnp.float32),
        scratch_shapes=[pltpu.VMEM((c, d), jnp.float32)],
        input_output_aliases=aliases,
        compiler_params=pltpu.CompilerParams(
            dimension_semantics=("parallel", "arbitrary"),
        ),
    )(*args)


def _sc_gather_rows(table, idx, *, chunk=32):
    v, w = table.shape
    bsz = idx.shape[0]
    info = plsc.get_sparse_core_info()
    nw = info.num_cores * info.num_subcores
    b_per_w = bsz // nw
    assert b_per_w * nw == bsz and b_per_w % chunk == 0
    nch = b_per_w // chunk
    mesh = plsc.VectorSubcoreMesh(core_axis_name="c", subcore_axis_name="s")

    out_type = [jax.ShapeDtypeStruct((bsz, w), jnp.float32)]
    scratch = [
        pltpu.VMEM((b_per_w,), jnp.int32),
        pltpu.VMEM((chunk, w), jnp.float32),
        pltpu.VMEM((chunk, w), jnp.float32),
        pltpu.SemaphoreType.DMA,
        pltpu.SemaphoreType.DMA,
        pltpu.SemaphoreType.DMA,
        pltpu.SemaphoreType.DMA,
    ]
    @functools.partial(pl.kernel, mesh=mesh, out_type=tuple(out_type),
                       scratch_types=scratch)
    def k(*refs):
        (table_hbm, idx_hbm, out_hbm, idx_v,
         r0, r1, sg0, sg1, sw0, sw1) = refs
        bufs = (r0, r1)
        gsems = (sg0, sg1)
        wsems = (sw0, sw1)
        wid = lax.axis_index("s") * info.num_cores + lax.axis_index("c")
        base = wid * b_per_w
        pltpu.sync_copy(idx_hbm.at[pl.ds(base, b_per_w)], idx_v)

        def start_gather(c):
            bi = c % 2
            return [pltpu.async_copy(
                table_hbm.at[idx_v.at[pl.ds(c * chunk, chunk)]],
                bufs[bi], gsems[bi])]

        def start_wb(c):
            bi = c % 2
            off = base + c * chunk
            return [pltpu.async_copy(
                bufs[bi], out_hbm.at[pl.ds(off, chunk)], wsems[bi])]

        pending_g = start_gather(0)
        pending_w = [None, None]
        for c in range(nch):
            bi = c % 2
            for cp in pending_g:
                cp.wait()
            if c + 1 < nch:
                if pending_w[1 - bi] is not None:
                    for cp in pending_w[1 - bi]:
                        cp.wait()
                    pending_w[1 - bi] = None
                pending_g = start_gather(c + 1)
            pending_w[bi] = start_wb(c)
        for pw in pending_w:
            if pw is not None:
                for cp in pw:
                    cp.wait()

    return k(table, idx)[0]


def _sc_return_scatter(eo_flat, src3, dst3, n_out, *, chunk=32):
    v, w = eo_flat.shape
    nw, nch, ck = src3.shape
    assert ck == chunk
    mesh = plsc.VectorSubcoreMesh(core_axis_name="c", subcore_axis_name="s")

    @functools.partial(
        pl.kernel, mesh=mesh,
        out_type=jax.ShapeDtypeStruct((n_out, w), jnp.float32),
        scratch_types=[
            pltpu.VMEM((nch, chunk), jnp.int32),
            pltpu.VMEM((nch, chunk), jnp.int32),
            pltpu.VMEM((chunk, w), jnp.float32),
            pltpu.VMEM((chunk, w), jnp.float32),
            pltpu.SemaphoreType.DMA,
            pltpu.SemaphoreType.DMA,
            pltpu.SemaphoreType.DMA,
            pltpu.SemaphoreType.DMA,
        ],
    )
    def k(eo_hbm, src_hbm, dst_hbm, out_hbm, src_v, dst_v,
          r0, r1, sg0, sg1, sw0, sw1):
        bufs = (r0, r1)
        gsems = (sg0, sg1)
        wsems = (sw0, sw1)
        wid = lax.axis_index("s") * 2 + lax.axis_index("c")
        pltpu.sync_copy(src_hbm.at[wid], src_v)
        pltpu.sync_copy(dst_hbm.at[wid], dst_v)

        def start_gather(c):
            bi = c % 2
            return pltpu.async_copy(
                eo_hbm.at[src_v.at[c]], bufs[bi], gsems[bi])

        def start_wb(c):
            bi = c % 2
            return pltpu.async_copy(
                bufs[bi], out_hbm.at[dst_v.at[c]], wsems[bi])

        pending_g = start_gather(0)
        pending_w = [None, None]
        for c in range(nch):
            bi = c % 2
            pending_g.wait()
            if c + 1 < nch:
                if pending_w[1 - bi] is not None:
                    pending_w[1 - bi].wait()
                    pending_w[1 - bi] = None
                pending_g = start_gather(c + 1)
            pending_w[bi] = start_wb(c)
        for pw in pending_w:
            if pw is not None:
                pw.wait()

    return k(eo_flat, src3, dst3)


def kernel(x, Wr, w1, b1, w2, b2):
    b, s, d = x.shape
    e = Wr.shape[0]
    x_flat = x.reshape(-1, d)
    n_tok = x_flat.shape[0]
    cap = int(1.25 * n_tok / e)

    router_logits = x_flat @ Wr.T
    router_probs = jax.nn.softmax(router_logits, axis=-1)
    gates = jnp.max(router_probs, axis=-1)
    indices = jnp.argmax(router_probs, axis=-1)

    expert_mask = jax.nn.one_hot(indices, e, dtype=jnp.float32)
    density = expert_mask.mean(axis=0)
    prob_mean = router_probs.mean(axis=0)
    load_balance_loss = e * jnp.sum(density * prob_mean) * 0.01
    router_z_loss = jnp.mean(
        jax.scipy.special.logsumexp(router_probs, axis=-1)) * 0.001
    aux_loss = load_balance_loss + router_z_loss

    tok = jnp.arange(n_tok, dtype=jnp.int32)
    idx32 = indices.astype(jnp.int32)
    sorted_e, sorted_negg, sorted_tok = jax.lax.sort(
        (idx32, -gates, tok), num_keys=2, is_stable=True)
    counts = jnp.sum(expert_mask, axis=0).astype(jnp.int32)
    seg_start = jnp.concatenate(
        [jnp.zeros((1,), jnp.int32), jnp.cumsum(counts)[:-1].astype(jnp.int32)])
    pos = tok - seg_start[sorted_e]
    keep = pos < cap
    fslot = sorted_e * cap + pos
    f_or_dummy = jnp.where(keep, fslot, e * cap)
    sel_gate = jnp.zeros((e * cap + 1,), jnp.float32).at[f_or_dummy].set(
        -sorted_negg)[: e * cap]

    e_star = jnp.argmin(counts).astype(jnp.int32)
    pad_slot = e_star * cap + counts[e_star]
    nw = 32
    disp_src3 = sorted_tok.reshape(nw, -1, 32)
    disp_dst3 = jnp.where(keep, fslot, pad_slot).reshape(nw, -1, 32)
    xe = _sc_return_scatter(x_flat, disp_src3, disp_dst3, e * cap).reshape(
        e, cap, d)
    gr = sel_gate.reshape(e, 1, cap)
    b1r = b1.reshape(e, -1, 1, 2048)
    b2r = b2.reshape(e, 1, d)

    eo = _ffn_half(xe, w1, b1r, w2, b2r, gr, None, dt=2048, e0=0)
    eo_flat = eo.reshape(e * cap, d)

    src_sorted = jnp.where(keep, fslot, pad_slot)
    src3 = src_sorted.reshape(nw, -1, 32)
    dst3 = sorted_tok.reshape(nw, -1, 32)
    out_flat = _sc_return_scatter(eo_flat, src3, dst3, n_tok)
    return out_flat.reshape(b, s, d), aux_loss
